# baseline (speedup 1.0000x reference)
import functools

import jax
import jax.numpy as jnp
from jax import lax
from jax.experimental import pallas as pl
from jax.experimental.pallas import tpu as pltpu
from jax.experimental.pallas import tpu_sc as plsc

N = 10000
NP = 10240
EP = 344064
B = 128
BW = 96
NBLK = 84
NBLK_CS = 224
NC, NS = 2, 16
NW = NC * NS
STRIPE = NP // NS
HALF = STRIPE // 2
DH = 64
BN_INV = float((1.0 + 1e-5) ** -0.5)

_MESH = plsc.VectorSubcoreMesh(core_axis_name="c", subcore_axis_name="s")


def _zero_fill(ref, nrows, ncols):
  z16 = jnp.zeros((16,), jnp.float32)
  def row(i, carry):
    for k in range(ncols // 16):
      ref[i, pl.ds(k * 16, 16)] = z16
    return carry
  lax.fori_loop(0, nrows, row, 0)


@functools.partial(
    pl.kernel,
    out_type=jax.ShapeDtypeStruct((NC, NP, DH), jnp.float32),
    mesh=_MESH,
    compiler_params=pltpu.CompilerParams(use_tc_tiling_on_sc=False),
    scratch_types=[
        pltpu.VMEM((NBLK_CS, BW), jnp.int32),
        pltpu.VMEM((NBLK_CS, BW), jnp.int32),
        [pltpu.VMEM((BW, DH), jnp.float32) for _ in range(4)],
        pltpu.VMEM((HALF, DH), jnp.float32),
        pltpu.VMEM_SHARED((NP, DH), jnp.float32),
        pltpu.SemaphoreType.DMA,
        pltpu.SemaphoreType.DMA,
    ],
)
def _agg128(table_l, table_r, srcb, dstb, out, src_v, dst_v, bufs,
            zbuf, acc, semg, sems):
  c = lax.axis_index("c")
  s = lax.axis_index("s")
  base = s * STRIPE

  _zero_fill(zbuf, HALF, DH)
  pltpu.sync_copy(zbuf, acc.at[pl.ds(base, HALF)])
  pltpu.sync_copy(zbuf, acc.at[pl.ds(base + HALF, HALF)])
  pltpu.sync_copy(srcb.at[s], src_v)
  pltpu.sync_copy(dstb.at[s], dst_v)
  plsc.subcore_barrier()

  def edge_loop(table):
    K = 2

    def drain_g(b):
      pltpu.make_async_copy(table.at[src_v.at[0]], bufs[b], semg).wait()

    def drain_s(b):
      pltpu.make_async_copy(bufs[b], acc.at[dst_v.at[0]], sems).wait()

    for b in range(K, 2 * K):
      _zero_fill(bufs[b], BW, DH)
    for b in range(K):
      pltpu.async_copy(table.at[src_v.at[b]], bufs[b], semg)
    for b in range(K, 2 * K):
      pltpu.async_copy(bufs[b], acc.at[dst_v.at[0]], sems, add=True)

    def body(it, carry):
      j0 = 2 * K * it
      for b in range(K):
        drain_g(b)
        pltpu.async_copy(bufs[b], acc.at[dst_v.at[j0 + b]], sems, add=True)
      for b in range(K):
        drain_s(K + b)
      for b in range(K):
        pltpu.async_copy(table.at[src_v.at[j0 + K + b]], bufs[K + b], semg)
      for b in range(K):
        drain_g(K + b)
        pltpu.async_copy(bufs[K + b], acc.at[dst_v.at[j0 + K + b]], sems,
                         add=True)
      for b in range(K):
        drain_s(b)
      for b in range(K):
        jn = jnp.minimum(j0 + 2 * K + b, NBLK_CS - 1)
        pltpu.async_copy(table.at[src_v.at[jn]], bufs[b], semg)
      return carry

    lax.fori_loop(0, NBLK_CS // (2 * K), body, 0)
    for b in range(K):
      drain_g(b)
      drain_s(K + b)

  @pl.when(c == 0)
  def _():
    edge_loop(table_l)

  @pl.when(c == 1)
  def _():
    edge_loop(table_r)

  plsc.subcore_barrier()

  pltpu.sync_copy(acc.at[pl.ds(base, HALF)], zbuf)
  pltpu.sync_copy(zbuf, out.at[c, pl.ds(base, HALF)])
  pltpu.sync_copy(acc.at[pl.ds(base + HALF, HALF)], zbuf)
  pltpu.sync_copy(zbuf, out.at[c, pl.ds(base + HALF, HALF)])


@functools.partial(
    pl.kernel,
    out_type=jax.ShapeDtypeStruct((NC, NP, 16), jnp.float32),
    mesh=_MESH,
    compiler_params=pltpu.CompilerParams(use_tc_tiling_on_sc=False),
    scratch_types=[
        pltpu.VMEM((NBLK, B), jnp.int32),
        pltpu.VMEM((NBLK, B), jnp.int32),
        pltpu.VMEM((B, 16), jnp.float32),
        pltpu.VMEM((B, 16), jnp.float32),
        pltpu.VMEM((HALF, 16), jnp.float32),
        pltpu.VMEM_SHARED((NP, 16), jnp.float32),
        pltpu.SemaphoreType.DMA,
    ],
)
def _agg16(table, srcb, dstb, out, src_v, dst_v, rows0, rows1, zbuf, acc,
           sem):
  c = lax.axis_index("c")
  s = lax.axis_index("s")
  wid = c * NS + s
  base = s * STRIPE

  _zero_fill(zbuf, HALF, 16)
  pltpu.sync_copy(zbuf, acc.at[pl.ds(base, HALF)])
  pltpu.sync_copy(zbuf, acc.at[pl.ds(base + HALF, HALF)])
  pltpu.sync_copy(srcb.at[wid], src_v)
  pltpu.sync_copy(dstb.at[wid], dst_v)
  plsc.subcore_barrier()

  pltpu.async_copy(table.at[src_v.at[0]], rows0, sem).wait()

  def body(i, carry):
    j = 2 * i
    cg = pltpu.async_copy(table.at[src_v.at[j + 1]], rows1, sem)
    pltpu.sync_copy(rows0, acc.at[dst_v.at[j]], add=True)
    cg.wait()
    jn = jnp.minimum(j + 2, NBLK - 1)
    cg2 = pltpu.async_copy(table.at[src_v.at[jn]], rows0, sem)
    pltpu.sync_copy(rows1, acc.at[dst_v.at[j + 1]], add=True)
    cg2.wait()
    return carry

  lax.fori_loop(0, NBLK // 2, body, 0)
  plsc.subcore_barrier()

  pltpu.sync_copy(acc.at[pl.ds(base, HALF)], zbuf)
  pltpu.sync_copy(zbuf, out.at[c, pl.ds(base, HALF)])
  pltpu.sync_copy(acc.at[pl.ds(base + HALF, HALF)], zbuf)
  pltpu.sync_copy(zbuf, out.at[c, pl.ds(base + HALF, HALF)])


@functools.partial(
    pl.kernel,
    out_type=jax.ShapeDtypeStruct((NC, NP, 16), jnp.float32),
    mesh=_MESH,
    compiler_params=pltpu.CompilerParams(use_tc_tiling_on_sc=False),
    scratch_types=[
        pltpu.VMEM((NBLK, B), jnp.int32),
        pltpu.VMEM((B, 16), jnp.float32),
        pltpu.VMEM((HALF, 16), jnp.float32),
        pltpu.VMEM_SHARED((NP, 16), jnp.float32),
    ],
)
def _deg_kernel(dstb, out, dst_v, ones_v, zbuf, acc):
  c = lax.axis_index("c")
  s = lax.axis_index("s")
  wid = c * NS + s
  base = s * STRIPE

  one16 = jnp.ones((16,), jnp.float32)
  def orow(i, carry):
    ones_v[i, pl.ds(0, 16)] = one16
    return carry
  lax.fori_loop(0, B, orow, 0)

  _zero_fill(zbuf, HALF, 16)
  pltpu.sync_copy(zbuf, acc.at[pl.ds(base, HALF)])
  pltpu.sync_copy(zbuf, acc.at[pl.ds(base + HALF, HALF)])
  pltpu.sync_copy(dstb.at[wid], dst_v)
  plsc.subcore_barrier()

  def body(j, carry):
    pltpu.sync_copy(ones_v, acc.at[dst_v.at[j]], add=True)
    return carry

  lax.fori_loop(0, NBLK, body, 0)
  plsc.subcore_barrier()

  pltpu.sync_copy(acc.at[pl.ds(base, HALF)], zbuf)
  pltpu.sync_copy(zbuf, out.at[c, pl.ds(base, HALF)])
  pltpu.sync_copy(acc.at[pl.ds(base + HALF, HALF)], zbuf)
  pltpu.sync_copy(zbuf, out.at[c, pl.ds(base + HALF, HALF)])


def _tc1_body(degp_ref, xp_ref, dis_ref, xl_ref, xr_ref):
  deg = degp_ref[0, :, 0:1] + degp_ref[1, :, 0:1]
  rows = lax.broadcasted_iota(jnp.int32, (NP, 1), 0)
  dis = jnp.where(rows < N, lax.rsqrt(jnp.maximum(deg, 1.0)), 0.0)
  dis_ref[...] = dis
  xs = dis * xp_ref[...]
  xl_ref[...] = xs[:, :DH]
  xr_ref[...] = xs[:, DH:]


_tc1 = pl.pallas_call(
    _tc1_body,
    out_shape=[jax.ShapeDtypeStruct((NP, 1), jnp.float32),
               jax.ShapeDtypeStruct((NP, DH), jnp.float32),
               jax.ShapeDtypeStruct((NP, DH), jnp.float32)],
)


def _tc2_body(p_ref, dis_ref, w0_ref, b0_ref, h0_ref, hl_ref, hr_ref):
  dis = dis_ref[...]
  a = jnp.concatenate([p_ref[0], p_ref[1]], axis=1)
  h = jnp.dot(dis * a, w0_ref[...],
              preferred_element_type=jnp.float32) + b0_ref[...]
  h = jnp.maximum(h, 0.0)
  h0_ref[...] = h
  hs = dis * h
  hl_ref[...] = hs[:, :DH]
  hr_ref[...] = hs[:, DH:]


_tc2 = pl.pallas_call(
    _tc2_body,
    out_shape=[jax.ShapeDtypeStruct((NP, 128), jnp.float32),
               jax.ShapeDtypeStruct((NP, DH), jnp.float32),
               jax.ShapeDtypeStruct((NP, DH), jnp.float32)],
)


def _tc3_body(p_ref, dis_ref, w1_ref, g_ref, b1_ref, bt_ref, h0_ref, w2p_ref,
              y2s_ref):
  dis = dis_ref[...]
  a = dis * jnp.concatenate([p_ref[0], p_ref[1]], axis=1)
  g = g_ref[...] * BN_INV
  w1g = w1_ref[...] * g
  b1g = b1_ref[...] * g + bt_ref[...]
  h1 = jnp.dot(a, w1g, preferred_element_type=jnp.float32) + b1g
  h1 = jnp.maximum(h1, 0.0) + h0_ref[...]
  y2s_ref[...] = dis * jnp.dot(h1, w2p_ref[...],
                               preferred_element_type=jnp.float32)


_tc3 = pl.pallas_call(
    _tc3_body,
    out_shape=jax.ShapeDtypeStruct((NP, 16), jnp.float32),
)


def _tc4_body(p_ref, dis_ref, b2p_ref, out_ref):
  z = dis_ref[...] * (p_ref[0] + p_ref[1]) + b2p_ref[...]
  z0 = z[:, 0:1]
  z1 = z[:, 1:2]
  m = jnp.maximum(z0, z1)
  lse = m + jnp.log(jnp.exp(z0 - m) + jnp.exp(z1 - m))
  out_ref[...] = z - lse


_tc4 = pl.pallas_call(
    _tc4_body,
    out_shape=jax.ShapeDtypeStruct((NP, 16), jnp.float32),
)


def kernel(x, edge_index, W0, b0, W1, b1, W2, b2, gamma1, beta1):
  ei = edge_index.astype(jnp.int32)
  loops = jnp.arange(N, dtype=jnp.int32)
  pad_n = EP - (ei.shape[1] + N)
  src = jnp.concatenate([ei[0], loops, jnp.zeros((pad_n,), jnp.int32)])
  dst = jnp.concatenate([ei[1], loops, jnp.full((pad_n,), NP - 1, jnp.int32)])
  srcb_e = src.reshape(NW, NBLK, B)
  dstb_e = dst.reshape(NW, NBLK, B)
  srcb_c = src.reshape(NS, NBLK_CS, BW)
  dstb_c = dst.reshape(NS, NBLK_CS, BW)
  xpad = jnp.pad(x, ((0, NP - N), (0, 0)))

  degp = _deg_kernel(dstb_e)
  dis, xl, xr = _tc1(degp, xpad)
  p1 = _agg128(xl, xr, srcb_c, dstb_c)
  h0, hl, hr = _tc2(p1, dis, W0, b0.reshape(1, -1))
  p2 = _agg128(hl, hr, srcb_c, dstb_c)
  w2p = jnp.pad(W2, ((0, 0), (0, 14)))
  y2s = _tc3(p2, dis, W1, gamma1.reshape(1, -1), b1.reshape(1, -1),
             beta1.reshape(1, -1), h0, w2p)
  p3 = _agg16(y2s, srcb_e, dstb_e)
  b2p = jnp.pad(b2, (0, 14)).reshape(1, 16)
  res = _tc4(p3, dis, b2p)
  return res[:N, :2]


# speedup vs baseline: 1.0051x; 1.0051x over previous
import functools

import jax
import jax.numpy as jnp
from jax import lax
from jax.experimental import pallas as pl
from jax.experimental.pallas import tpu as pltpu
from jax.experimental.pallas import tpu_sc as plsc

N = 10000
NP = 10240
EP = 344064
B = 128
BW = 96
NBLK = 84
NBLK_CS = 224
NC, NS = 2, 16
NW = NC * NS
STRIPE = NP // NS
HALF = STRIPE // 2
DH = 64
BN_INV = float((1.0 + 1e-5) ** -0.5)

_MESH = plsc.VectorSubcoreMesh(core_axis_name="c", subcore_axis_name="s")


def _zero_fill(ref, nrows, ncols):
  z16 = jnp.zeros((16,), jnp.float32)
  def row(i, carry):
    for k in range(ncols // 16):
      ref[i, pl.ds(k * 16, 16)] = z16
    return carry
  lax.fori_loop(0, nrows, row, 0)


@functools.partial(
    pl.kernel,
    out_type=jax.ShapeDtypeStruct((NC, NP, DH), jnp.float32),
    mesh=_MESH,
    compiler_params=pltpu.CompilerParams(use_tc_tiling_on_sc=False),
    scratch_types=[
        pltpu.VMEM((NBLK_CS, BW), jnp.int32),
        pltpu.VMEM((NBLK_CS, BW), jnp.int32),
        [pltpu.VMEM((BW, DH), jnp.float32) for _ in range(4)],
        pltpu.VMEM((HALF, DH), jnp.float32),
        pltpu.VMEM_SHARED((NP, DH), jnp.float32),
        pltpu.SemaphoreType.DMA,
        pltpu.SemaphoreType.DMA,
    ],
)
def _agg128(table_l, table_r, srcb, dstb, out, src_v, dst_v, bufs,
            zbuf, acc, semg, sems):
  c = lax.axis_index("c")
  s = lax.axis_index("s")
  base = s * STRIPE

  _zero_fill(zbuf, HALF, DH)
  pltpu.sync_copy(zbuf, acc.at[pl.ds(base, HALF)])
  pltpu.sync_copy(zbuf, acc.at[pl.ds(base + HALF, HALF)])
  pltpu.sync_copy(srcb.at[s], src_v)
  pltpu.sync_copy(dstb.at[s], dst_v)
  plsc.subcore_barrier()

  def edge_loop(table):
    K = 2

    def drain_g(b):
      pltpu.make_async_copy(table.at[src_v.at[0]], bufs[b], semg).wait()

    def drain_s(b):
      pltpu.make_async_copy(bufs[b], acc.at[dst_v.at[0]], sems).wait()

    for b in range(K, 2 * K):
      _zero_fill(bufs[b], BW, DH)
    for b in range(K):
      pltpu.async_copy(table.at[src_v.at[b]], bufs[b], semg)
    for b in range(K, 2 * K):
      pltpu.async_copy(bufs[b], acc.at[dst_v.at[0]], sems, add=True)

    def body(it, carry):
      j0 = 2 * K * it
      for b in range(K):
        drain_g(b)
        pltpu.async_copy(bufs[b], acc.at[dst_v.at[j0 + b]], sems, add=True)
      for b in range(K):
        drain_s(K + b)
      for b in range(K):
        pltpu.async_copy(table.at[src_v.at[j0 + K + b]], bufs[K + b], semg)
      for b in range(K):
        drain_g(K + b)
        pltpu.async_copy(bufs[K + b], acc.at[dst_v.at[j0 + K + b]], sems,
                         add=True)
      for b in range(K):
        drain_s(b)
      for b in range(K):
        jn = jnp.minimum(j0 + 2 * K + b, NBLK_CS - 1)
        pltpu.async_copy(table.at[src_v.at[jn]], bufs[b], semg)
      return carry

    lax.fori_loop(0, NBLK_CS // (2 * K), body, 0)
    for b in range(K):
      drain_g(b)
      drain_s(K + b)

  @pl.when(c == 0)
  def _():
    edge_loop(table_l)

  @pl.when(c == 1)
  def _():
    edge_loop(table_r)

  plsc.subcore_barrier()

  pltpu.sync_copy(acc.at[pl.ds(base, HALF)], zbuf)
  pltpu.sync_copy(zbuf, out.at[c, pl.ds(base, HALF)])
  pltpu.sync_copy(acc.at[pl.ds(base + HALF, HALF)], zbuf)
  pltpu.sync_copy(zbuf, out.at[c, pl.ds(base + HALF, HALF)])


@functools.partial(
    pl.kernel,
    out_type=jax.ShapeDtypeStruct((NC, NP, 16), jnp.float32),
    mesh=_MESH,
    compiler_params=pltpu.CompilerParams(use_tc_tiling_on_sc=False),
    scratch_types=[
        pltpu.VMEM((NBLK, B), jnp.int32),
        pltpu.VMEM((NBLK, B), jnp.int32),
        pltpu.VMEM((B, 16), jnp.float32),
        pltpu.VMEM((B, 16), jnp.float32),
        pltpu.VMEM((HALF, 16), jnp.float32),
        pltpu.VMEM_SHARED((NP, 16), jnp.float32),
        pltpu.SemaphoreType.DMA,
    ],
)
def _agg16(table, srcb, dstb, out, src_v, dst_v, rows0, rows1, zbuf, acc,
           sem):
  c = lax.axis_index("c")
  s = lax.axis_index("s")
  wid = c * NS + s
  base = s * STRIPE

  _zero_fill(zbuf, HALF, 16)
  pltpu.sync_copy(zbuf, acc.at[pl.ds(base, HALF)])
  pltpu.sync_copy(zbuf, acc.at[pl.ds(base + HALF, HALF)])
  pltpu.sync_copy(srcb.at[wid], src_v)
  pltpu.sync_copy(dstb.at[wid], dst_v)
  plsc.subcore_barrier()

  pltpu.async_copy(table.at[src_v.at[0]], rows0, sem).wait()

  def body(i, carry):
    j = 2 * i
    cg = pltpu.async_copy(table.at[src_v.at[j + 1]], rows1, sem)
    pltpu.sync_copy(rows0, acc.at[dst_v.at[j]], add=True)
    cg.wait()
    jn = jnp.minimum(j + 2, NBLK - 1)
    cg2 = pltpu.async_copy(table.at[src_v.at[jn]], rows0, sem)
    pltpu.sync_copy(rows1, acc.at[dst_v.at[j + 1]], add=True)
    cg2.wait()
    return carry

  lax.fori_loop(0, NBLK // 2, body, 0)
  plsc.subcore_barrier()

  pltpu.sync_copy(acc.at[pl.ds(base, HALF)], zbuf)
  pltpu.sync_copy(zbuf, out.at[c, pl.ds(base, HALF)])
  pltpu.sync_copy(acc.at[pl.ds(base + HALF, HALF)], zbuf)
  pltpu.sync_copy(zbuf, out.at[c, pl.ds(base + HALF, HALF)])


@functools.partial(
    pl.kernel,
    out_type=jax.ShapeDtypeStruct((NC, NP, 16), jnp.float32),
    mesh=_MESH,
    compiler_params=pltpu.CompilerParams(use_tc_tiling_on_sc=False),
    scratch_types=[
        pltpu.VMEM((NBLK, B), jnp.int32),
        pltpu.VMEM((B, 16), jnp.float32),
        pltpu.VMEM((HALF, 16), jnp.float32),
        pltpu.VMEM_SHARED((NP, 16), jnp.float32),
    ],
)
def _deg_kernel(dstb, out, dst_v, ones_v, zbuf, acc):
  c = lax.axis_index("c")
  s = lax.axis_index("s")
  wid = c * NS + s
  base = s * STRIPE

  one16 = jnp.ones((16,), jnp.float32)
  def orow(i, carry):
    ones_v[i, pl.ds(0, 16)] = one16
    return carry
  lax.fori_loop(0, B, orow, 0)

  _zero_fill(zbuf, HALF, 16)
  pltpu.sync_copy(zbuf, acc.at[pl.ds(base, HALF)])
  pltpu.sync_copy(zbuf, acc.at[pl.ds(base + HALF, HALF)])
  pltpu.sync_copy(dstb.at[wid], dst_v)
  plsc.subcore_barrier()

  def body(j, carry):
    pltpu.sync_copy(ones_v, acc.at[dst_v.at[j]], add=True)
    return carry

  lax.fori_loop(0, NBLK, body, 0)
  plsc.subcore_barrier()

  pltpu.sync_copy(acc.at[pl.ds(base, HALF)], zbuf)
  pltpu.sync_copy(zbuf, out.at[c, pl.ds(base, HALF)])
  pltpu.sync_copy(acc.at[pl.ds(base + HALF, HALF)], zbuf)
  pltpu.sync_copy(zbuf, out.at[c, pl.ds(base + HALF, HALF)])


def _tc1_body(degp_ref, xp_ref, dis_ref, xl_ref, xr_ref):
  deg = degp_ref[0, :, 0:1] + degp_ref[1, :, 0:1]
  rows = lax.broadcasted_iota(jnp.int32, (NP, 1), 0)
  dis = jnp.where(rows < N, lax.rsqrt(jnp.maximum(deg, 1.0)), 0.0)
  dis_ref[...] = dis
  xs = dis * xp_ref[...]
  xl_ref[...] = xs[:, :DH]
  xr_ref[...] = xs[:, DH:]


_tc1 = pl.pallas_call(
    _tc1_body,
    out_shape=[jax.ShapeDtypeStruct((NP, 1), jnp.float32),
               jax.ShapeDtypeStruct((NP, DH), jnp.float32),
               jax.ShapeDtypeStruct((NP, DH), jnp.float32)],
)


def _tc2_body(p_ref, dis_ref, w0_ref, b0_ref, h0_ref, hl_ref, hr_ref):
  dis = dis_ref[...]
  a = jnp.concatenate([p_ref[0], p_ref[1]], axis=1)
  h = jnp.dot(dis * a, w0_ref[...],
              preferred_element_type=jnp.float32) + b0_ref[...]
  h = jnp.maximum(h, 0.0)
  h0_ref[...] = h
  hs = dis * h
  hl_ref[...] = hs[:, :DH]
  hr_ref[...] = hs[:, DH:]


_tc2 = pl.pallas_call(
    _tc2_body,
    out_shape=[jax.ShapeDtypeStruct((NP, 128), jnp.float32),
               jax.ShapeDtypeStruct((NP, DH), jnp.float32),
               jax.ShapeDtypeStruct((NP, DH), jnp.float32)],
)


def _tc3_body(p_ref, dis_ref, w1_ref, g_ref, b1_ref, bt_ref, h0_ref, w2p_ref,
              y2s_ref):
  dis = dis_ref[...]
  a = dis * jnp.concatenate([p_ref[0], p_ref[1]], axis=1)
  g = g_ref[...] * BN_INV
  w1g = w1_ref[...] * g
  b1g = b1_ref[...] * g + bt_ref[...]
  h1 = jnp.dot(a, w1g, preferred_element_type=jnp.float32) + b1g
  h1 = jnp.maximum(h1, 0.0) + h0_ref[...]
  y2s_ref[...] = dis * jnp.dot(h1, w2p_ref[...],
                               preferred_element_type=jnp.float32)


_tc3 = pl.pallas_call(
    _tc3_body,
    out_shape=jax.ShapeDtypeStruct((NP, 16), jnp.float32),
)


def _tc4_body(p_ref, dis_ref, b2p_ref, out_ref):
  z = dis_ref[...] * (p_ref[0] + p_ref[1]) + b2p_ref[...]
  z0 = z[:, 0:1]
  z1 = z[:, 1:2]
  m = jnp.maximum(z0, z1)
  lse = m + jnp.log(jnp.exp(z0 - m) + jnp.exp(z1 - m))
  out_ref[...] = z - lse


_tc4 = pl.pallas_call(
    _tc4_body,
    out_shape=jax.ShapeDtypeStruct((NP, 16), jnp.float32),
)


def kernel(x, edge_index, W0, b0, W1, b1, W2, b2, gamma1, beta1):
  ei = edge_index.astype(jnp.int32)
  loops = jnp.arange(N, dtype=jnp.int32)
  pad_n = EP - (ei.shape[1] + N)
  src = jnp.concatenate([ei[0], loops, jnp.zeros((pad_n,), jnp.int32)])
  pad_dst = N + (jnp.arange(pad_n, dtype=jnp.int32) % (NP - N))
  dst = jnp.concatenate([ei[1], loops, pad_dst])
  srcb_e = src.reshape(NW, NBLK, B)
  dstb_e = dst.reshape(NW, NBLK, B)
  srcb_c = src.reshape(NS, NBLK_CS, BW)
  dstb_c = dst.reshape(NS, NBLK_CS, BW)
  xpad = jnp.pad(x, ((0, NP - N), (0, 0)))

  degp = _deg_kernel(dstb_e)
  dis, xl, xr = _tc1(degp, xpad)
  p1 = _agg128(xl, xr, srcb_c, dstb_c)
  h0, hl, hr = _tc2(p1, dis, W0, b0.reshape(1, -1))
  p2 = _agg128(hl, hr, srcb_c, dstb_c)
  w2p = jnp.pad(W2, ((0, 0), (0, 14)))
  y2s = _tc3(p2, dis, W1, gamma1.reshape(1, -1), b1.reshape(1, -1),
             beta1.reshape(1, -1), h0, w2p)
  p3 = _agg16(y2s, srcb_e, dstb_e)
  b2p = jnp.pad(b2, (0, 14)).reshape(1, 16)
  res = _tc4(p3, dis, b2p)
  return res[:N, :2]


# speedup vs baseline: 2.3145x; 2.3027x over previous
import functools

import jax
import jax.numpy as jnp
from jax import lax
from jax.experimental import pallas as pl
from jax.experimental.pallas import tpu as pltpu
from jax.experimental.pallas import tpu_sc as plsc

N = 10000
NP = 10240
EP = 344064
B = 128
BW = 96
NBLK = 84
NBLK_CS = 224
NC, NS = 2, 16
NW = NC * NS
STRIPE = NP // NS
HALF = STRIPE // 2
DH = 64
BN_INV = float((1.0 + 1e-5) ** -0.5)

_MESH = plsc.VectorSubcoreMesh(core_axis_name="c", subcore_axis_name="s")


def _zero_fill(ref, nrows, ncols):
  z16 = jnp.zeros((16,), jnp.float32)
  def row(i, carry):
    for k in range(ncols // 16):
      ref[i, pl.ds(k * 16, 16)] = z16
    return carry
  lax.fori_loop(0, nrows, row, 0)


@functools.partial(
    pl.kernel,
    out_type=jax.ShapeDtypeStruct((NC, NP, DH), jnp.float32),
    mesh=_MESH,
    compiler_params=pltpu.CompilerParams(use_tc_tiling_on_sc=False),
    scratch_types=[
        pltpu.VMEM((NBLK_CS, BW), jnp.int32),
        pltpu.VMEM((NBLK_CS, BW), jnp.int32),
        [pltpu.VMEM((BW, DH), jnp.float32) for _ in range(4)],
        pltpu.VMEM((HALF, DH), jnp.float32),
        pltpu.VMEM_SHARED((NP, DH), jnp.float32),
        pltpu.SemaphoreType.DMA,
        pltpu.SemaphoreType.DMA,
    ],
)
def _agg128(table_l, table_r, srcb, dstb, out, src_v, dst_v, bufs,
            zbuf, acc, semg, sems):
  c = lax.axis_index("c")
  s = lax.axis_index("s")
  base = s * STRIPE

  _zero_fill(zbuf, HALF, DH)
  pltpu.sync_copy(zbuf, acc.at[pl.ds(base, HALF)])
  pltpu.sync_copy(zbuf, acc.at[pl.ds(base + HALF, HALF)])
  pltpu.sync_copy(srcb.at[s], src_v)
  pltpu.sync_copy(dstb.at[s], dst_v)
  plsc.subcore_barrier()

  def edge_loop(table):
    K = 2

    def drain_g(b):
      pltpu.make_async_copy(table.at[src_v.at[0]], bufs[b], semg).wait()

    def drain_s(b):
      pltpu.make_async_copy(bufs[b], acc.at[dst_v.at[0]], sems).wait()

    for b in range(K, 2 * K):
      _zero_fill(bufs[b], BW, DH)
    for b in range(K):
      pltpu.async_copy(table.at[src_v.at[b]], bufs[b], semg)
    for b in range(K, 2 * K):
      pltpu.async_copy(bufs[b], acc.at[dst_v.at[0]], sems, add=True)

    def body(it, carry):
      j0 = 2 * K * it
      for b in range(K):
        drain_g(b)
        pltpu.async_copy(bufs[b], acc.at[dst_v.at[j0 + b]], sems, add=True)
      for b in range(K):
        drain_s(K + b)
      for b in range(K):
        pltpu.async_copy(table.at[src_v.at[j0 + K + b]], bufs[K + b], semg)
      for b in range(K):
        drain_g(K + b)
        pltpu.async_copy(bufs[K + b], acc.at[dst_v.at[j0 + K + b]], sems,
                         add=True)
      for b in range(K):
        drain_s(b)
      for b in range(K):
        jn = jnp.minimum(j0 + 2 * K + b, NBLK_CS - 1)
        pltpu.async_copy(table.at[src_v.at[jn]], bufs[b], semg)
      return carry

    lax.fori_loop(0, NBLK_CS // (2 * K), body, 0)
    for b in range(K):
      drain_g(b)
      drain_s(K + b)

  @pl.when(c == 0)
  def _():
    edge_loop(table_l)

  @pl.when(c == 1)
  def _():
    edge_loop(table_r)

  plsc.subcore_barrier()

  pltpu.sync_copy(acc.at[pl.ds(base, HALF)], zbuf)
  pltpu.sync_copy(zbuf, out.at[c, pl.ds(base, HALF)])
  pltpu.sync_copy(acc.at[pl.ds(base + HALF, HALF)], zbuf)
  pltpu.sync_copy(zbuf, out.at[c, pl.ds(base + HALF, HALF)])


@functools.partial(
    pl.kernel,
    out_type=jax.ShapeDtypeStruct((NC, NP, 16), jnp.float32),
    mesh=_MESH,
    compiler_params=pltpu.CompilerParams(use_tc_tiling_on_sc=False),
    scratch_types=[
        pltpu.VMEM((NBLK, B), jnp.int32),
        pltpu.VMEM((NBLK, B), jnp.int32),
        pltpu.VMEM((B, 16), jnp.float32),
        pltpu.VMEM((B, 16), jnp.float32),
        pltpu.VMEM((HALF, 16), jnp.float32),
        pltpu.VMEM_SHARED((NP, 16), jnp.float32),
        pltpu.SemaphoreType.DMA,
    ],
)
def _agg16(table, srcb, dstb, out, src_v, dst_v, rows0, rows1, zbuf, acc,
           sem):
  c = lax.axis_index("c")
  s = lax.axis_index("s")
  wid = c * NS + s
  base = s * STRIPE

  _zero_fill(zbuf, HALF, 16)
  pltpu.sync_copy(zbuf, acc.at[pl.ds(base, HALF)])
  pltpu.sync_copy(zbuf, acc.at[pl.ds(base + HALF, HALF)])
  pltpu.sync_copy(srcb.at[wid], src_v)
  pltpu.sync_copy(dstb.at[wid], dst_v)
  plsc.subcore_barrier()

  pltpu.async_copy(table.at[src_v.at[0]], rows0, sem).wait()

  def body(i, carry):
    j = 2 * i
    cg = pltpu.async_copy(table.at[src_v.at[j + 1]], rows1, sem)
    pltpu.sync_copy(rows0, acc.at[dst_v.at[j]], add=True)
    cg.wait()
    jn = jnp.minimum(j + 2, NBLK - 1)
    cg2 = pltpu.async_copy(table.at[src_v.at[jn]], rows0, sem)
    pltpu.sync_copy(rows1, acc.at[dst_v.at[j + 1]], add=True)
    cg2.wait()
    return carry

  lax.fori_loop(0, NBLK // 2, body, 0)
  plsc.subcore_barrier()

  pltpu.sync_copy(acc.at[pl.ds(base, HALF)], zbuf)
  pltpu.sync_copy(zbuf, out.at[c, pl.ds(base, HALF)])
  pltpu.sync_copy(acc.at[pl.ds(base + HALF, HALF)], zbuf)
  pltpu.sync_copy(zbuf, out.at[c, pl.ds(base + HALF, HALF)])


@functools.partial(
    pl.kernel,
    out_type=jax.ShapeDtypeStruct((NC, NP, 16), jnp.float32),
    mesh=_MESH,
    compiler_params=pltpu.CompilerParams(use_tc_tiling_on_sc=False),
    scratch_types=[
        pltpu.VMEM((NBLK, B), jnp.int32),
        pltpu.VMEM((B, 16), jnp.float32),
        pltpu.VMEM((HALF, 16), jnp.float32),
        pltpu.VMEM_SHARED((NP, 16), jnp.float32),
    ],
)
def _deg_kernel(dstb, out, dst_v, ones_v, zbuf, acc):
  c = lax.axis_index("c")
  s = lax.axis_index("s")
  wid = c * NS + s
  base = s * STRIPE

  one16 = jnp.ones((16,), jnp.float32)
  def orow(i, carry):
    ones_v[i, pl.ds(0, 16)] = one16
    return carry
  lax.fori_loop(0, B, orow, 0)

  _zero_fill(zbuf, HALF, 16)
  pltpu.sync_copy(zbuf, acc.at[pl.ds(base, HALF)])
  pltpu.sync_copy(zbuf, acc.at[pl.ds(base + HALF, HALF)])
  pltpu.sync_copy(dstb.at[wid], dst_v)
  plsc.subcore_barrier()

  def body(j, carry):
    pltpu.sync_copy(ones_v, acc.at[dst_v.at[j]], add=True)
    return carry

  lax.fori_loop(0, NBLK, body, 0)
  plsc.subcore_barrier()

  pltpu.sync_copy(acc.at[pl.ds(base, HALF)], zbuf)
  pltpu.sync_copy(zbuf, out.at[c, pl.ds(base, HALF)])
  pltpu.sync_copy(acc.at[pl.ds(base + HALF, HALF)], zbuf)
  pltpu.sync_copy(zbuf, out.at[c, pl.ds(base + HALF, HALF)])


def _tc1_body(degp_ref, xp_ref, dis_ref, xl_ref, xr_ref):
  deg = degp_ref[0, :, 0:1] + degp_ref[1, :, 0:1]
  rows = lax.broadcasted_iota(jnp.int32, (NP, 1), 0)
  dis = jnp.where(rows < N, lax.rsqrt(jnp.maximum(deg, 1.0)), 0.0)
  dis_ref[...] = dis
  xs = dis * xp_ref[...]
  xl_ref[...] = xs[:, :DH]
  xr_ref[...] = xs[:, DH:]


_tc1 = pl.pallas_call(
    _tc1_body,
    out_shape=[jax.ShapeDtypeStruct((NP, 1), jnp.float32),
               jax.ShapeDtypeStruct((NP, DH), jnp.float32),
               jax.ShapeDtypeStruct((NP, DH), jnp.float32)],
)


def _tc2_body(p_ref, dis_ref, w0_ref, b0_ref, h0_ref, hl_ref, hr_ref):
  dis = dis_ref[...]
  a = jnp.concatenate([p_ref[0], p_ref[1]], axis=1)
  h = jnp.dot(dis * a, w0_ref[...],
              preferred_element_type=jnp.float32) + b0_ref[...]
  h = jnp.maximum(h, 0.0)
  h0_ref[...] = h
  hs = dis * h
  hl_ref[...] = hs[:, :DH]
  hr_ref[...] = hs[:, DH:]


_tc2 = pl.pallas_call(
    _tc2_body,
    out_shape=[jax.ShapeDtypeStruct((NP, 128), jnp.float32),
               jax.ShapeDtypeStruct((NP, DH), jnp.float32),
               jax.ShapeDtypeStruct((NP, DH), jnp.float32)],
)


def _tc3_body(p_ref, dis_ref, w1_ref, g_ref, b1_ref, bt_ref, h0_ref, w2p_ref,
              y2s_ref):
  dis = dis_ref[...]
  a = dis * jnp.concatenate([p_ref[0], p_ref[1]], axis=1)
  g = g_ref[...] * BN_INV
  w1g = w1_ref[...] * g
  b1g = b1_ref[...] * g + bt_ref[...]
  h1 = jnp.dot(a, w1g, preferred_element_type=jnp.float32) + b1g
  h1 = jnp.maximum(h1, 0.0) + h0_ref[...]
  y2s_ref[...] = dis * jnp.dot(h1, w2p_ref[...],
                               preferred_element_type=jnp.float32)


_tc3 = pl.pallas_call(
    _tc3_body,
    out_shape=jax.ShapeDtypeStruct((NP, 16), jnp.float32),
)


def _tc4_body(p_ref, dis_ref, b2p_ref, out_ref):
  z = dis_ref[...] * (p_ref[0] + p_ref[1]) + b2p_ref[...]
  z0 = z[:, 0:1]
  z1 = z[:, 1:2]
  m = jnp.maximum(z0, z1)
  lse = m + jnp.log(jnp.exp(z0 - m) + jnp.exp(z1 - m))
  out_ref[...] = z - lse


_tc4 = pl.pallas_call(
    _tc4_body,
    out_shape=jax.ShapeDtypeStruct((NP, 16), jnp.float32),
)


def kernel(x, edge_index, W0, b0, W1, b1, W2, b2, gamma1, beta1):
  ei = edge_index.astype(jnp.int32)
  loops = jnp.arange(N, dtype=jnp.int32)
  pad_n = EP - (ei.shape[1] + N)
  pad_src = jnp.arange(pad_n, dtype=jnp.int32) % N
  src = jnp.concatenate([ei[0], loops, pad_src])
  pad_dst = N + (jnp.arange(pad_n, dtype=jnp.int32) % (NP - N))
  dst = jnp.concatenate([ei[1], loops, pad_dst])
  srcb_e = src.reshape(NW, NBLK, B)
  dstb_e = dst.reshape(NW, NBLK, B)
  srcb_c = src.reshape(NS, NBLK_CS, BW)
  dstb_c = dst.reshape(NS, NBLK_CS, BW)
  xpad = jnp.pad(x, ((0, NP - N), (0, 0)))

  degp = _deg_kernel(dstb_e)
  dis, xl, xr = _tc1(degp, xpad)
  p1 = _agg128(xl, xr, srcb_c, dstb_c)
  h0, hl, hr = _tc2(p1, dis, W0, b0.reshape(1, -1))
  p2 = _agg128(hl, hr, srcb_c, dstb_c)
  w2p = jnp.pad(W2, ((0, 0), (0, 14)))
  y2s = _tc3(p2, dis, W1, gamma1.reshape(1, -1), b1.reshape(1, -1),
             beta1.reshape(1, -1), h0, w2p)
  p3 = _agg16(y2s, srcb_e, dstb_e)
  b2p = jnp.pad(b2, (0, 14)).reshape(1, 16)
  res = _tc4(p3, dis, b2p)
  return res[:N, :2]


# speedup vs baseline: 2.4129x; 1.0425x over previous
import functools

import jax
import jax.numpy as jnp
from jax import lax
from jax.experimental import pallas as pl
from jax.experimental.pallas import tpu as pltpu
from jax.experimental.pallas import tpu_sc as plsc

N = 10000
NP = 10240
EP = 344064
B = 128
BW = 96
NBLK = 84
NBLK_CS = 224
NC, NS = 2, 16
NW = NC * NS
STRIPE = NP // NS
HALF = STRIPE // 2
DH = 64
BN_INV = float((1.0 + 1e-5) ** -0.5)

_MESH = plsc.VectorSubcoreMesh(core_axis_name="c", subcore_axis_name="s")


def _zero_fill(ref, nrows, ncols):
  z16 = jnp.zeros((16,), jnp.float32)
  def row(i, carry):
    for k in range(ncols // 16):
      ref[i, pl.ds(k * 16, 16)] = z16
    return carry
  lax.fori_loop(0, nrows, row, 0)


@functools.partial(
    pl.kernel,
    out_type=jax.ShapeDtypeStruct((NC, NP, DH), jnp.float32),
    mesh=_MESH,
    compiler_params=pltpu.CompilerParams(use_tc_tiling_on_sc=False),
    scratch_types=[
        pltpu.VMEM((NBLK_CS, BW), jnp.int32),
        pltpu.VMEM((NBLK_CS, BW), jnp.int32),
        [pltpu.VMEM((BW, DH), jnp.float32) for _ in range(4)],
        pltpu.VMEM((HALF, DH), jnp.float32),
        pltpu.VMEM_SHARED((NP, DH), jnp.float32),
        pltpu.SemaphoreType.DMA,
        pltpu.SemaphoreType.DMA,
    ],
)
def _agg128(table_l, table_r, srcb, dstb, out, src_v, dst_v, bufs,
            zbuf, acc, semg, sems):
  c = lax.axis_index("c")
  s = lax.axis_index("s")
  base = s * STRIPE

  _zero_fill(zbuf, HALF, DH)
  pltpu.sync_copy(zbuf, acc.at[pl.ds(base, HALF)])
  pltpu.sync_copy(zbuf, acc.at[pl.ds(base + HALF, HALF)])
  pltpu.sync_copy(srcb.at[s], src_v)
  pltpu.sync_copy(dstb.at[s], dst_v)
  plsc.subcore_barrier()

  def edge_loop(table):
    K = 2

    def drain_g(b):
      pltpu.make_async_copy(table.at[src_v.at[0]], bufs[b], semg).wait()

    def drain_s(b):
      pltpu.make_async_copy(bufs[b], acc.at[dst_v.at[0]], sems).wait()

    for b in range(K, 2 * K):
      _zero_fill(bufs[b], BW, DH)
    for b in range(K):
      pltpu.async_copy(table.at[src_v.at[b]], bufs[b], semg)
    for b in range(K, 2 * K):
      pltpu.async_copy(bufs[b], acc.at[dst_v.at[0]], sems, add=True)

    def body(it, carry):
      j0 = 2 * K * it
      for b in range(K):
        drain_g(b)
        pltpu.async_copy(bufs[b], acc.at[dst_v.at[j0 + b]], sems, add=True)
      for b in range(K):
        drain_s(K + b)
      for b in range(K):
        pltpu.async_copy(table.at[src_v.at[j0 + K + b]], bufs[K + b], semg)
      for b in range(K):
        drain_g(K + b)
        pltpu.async_copy(bufs[K + b], acc.at[dst_v.at[j0 + K + b]], sems,
                         add=True)
      for b in range(K):
        drain_s(b)
      for b in range(K):
        jn = jnp.minimum(j0 + 2 * K + b, NBLK_CS - 1)
        pltpu.async_copy(table.at[src_v.at[jn]], bufs[b], semg)
      return carry

    lax.fori_loop(0, NBLK_CS // (2 * K), body, 0)
    for b in range(K):
      drain_g(b)
      drain_s(K + b)

  @pl.when(c == 0)
  def _():
    edge_loop(table_l)

  @pl.when(c == 1)
  def _():
    edge_loop(table_r)

  plsc.subcore_barrier()

  pltpu.sync_copy(acc.at[pl.ds(base, HALF)], zbuf)
  pltpu.sync_copy(zbuf, out.at[c, pl.ds(base, HALF)])
  pltpu.sync_copy(acc.at[pl.ds(base + HALF, HALF)], zbuf)
  pltpu.sync_copy(zbuf, out.at[c, pl.ds(base + HALF, HALF)])


@functools.partial(
    pl.kernel,
    out_type=jax.ShapeDtypeStruct((NC, NP, 16), jnp.float32),
    mesh=_MESH,
    compiler_params=pltpu.CompilerParams(use_tc_tiling_on_sc=False),
    scratch_types=[
        pltpu.VMEM((NBLK, B), jnp.int32),
        pltpu.VMEM((NBLK, B), jnp.int32),
        [pltpu.VMEM((B, 16), jnp.float32) for _ in range(4)],
        pltpu.VMEM((HALF, 16), jnp.float32),
        pltpu.VMEM_SHARED((NP, 16), jnp.float32),
        pltpu.SemaphoreType.DMA,
        pltpu.SemaphoreType.DMA,
    ],
)
def _agg16(table, srcb, dstb, out, src_v, dst_v, bufs, zbuf, acc, semg,
           sems):
  c = lax.axis_index("c")
  s = lax.axis_index("s")
  wid = c * NS + s
  base = s * STRIPE

  _zero_fill(zbuf, HALF, 16)
  pltpu.sync_copy(zbuf, acc.at[pl.ds(base, HALF)])
  pltpu.sync_copy(zbuf, acc.at[pl.ds(base + HALF, HALF)])
  pltpu.sync_copy(srcb.at[wid], src_v)
  pltpu.sync_copy(dstb.at[wid], dst_v)
  plsc.subcore_barrier()

  K = 2

  def drain_g(b):
    pltpu.make_async_copy(table.at[src_v.at[0]], bufs[b], semg).wait()

  def drain_s(b):
    pltpu.make_async_copy(bufs[b], acc.at[dst_v.at[0]], sems).wait()

  for b in range(K, 2 * K):
    _zero_fill(bufs[b], B, 16)
  for b in range(K):
    pltpu.async_copy(table.at[src_v.at[b]], bufs[b], semg)
  for b in range(K, 2 * K):
    pltpu.async_copy(bufs[b], acc.at[dst_v.at[0]], sems, add=True)

  def body(it, carry):
    j0 = 2 * K * it
    for b in range(K):
      drain_g(b)
      pltpu.async_copy(bufs[b], acc.at[dst_v.at[j0 + b]], sems, add=True)
    for b in range(K):
      drain_s(K + b)
    for b in range(K):
      pltpu.async_copy(table.at[src_v.at[j0 + K + b]], bufs[K + b], semg)
    for b in range(K):
      drain_g(K + b)
      pltpu.async_copy(bufs[K + b], acc.at[dst_v.at[j0 + K + b]], sems,
                       add=True)
    for b in range(K):
      drain_s(b)
    for b in range(K):
      jn = jnp.minimum(j0 + 2 * K + b, NBLK - 1)
      pltpu.async_copy(table.at[src_v.at[jn]], bufs[b], semg)
    return carry

  lax.fori_loop(0, NBLK // (2 * K), body, 0)
  for b in range(K):
    drain_g(b)
    drain_s(K + b)
  plsc.subcore_barrier()

  pltpu.sync_copy(acc.at[pl.ds(base, HALF)], zbuf)
  pltpu.sync_copy(zbuf, out.at[c, pl.ds(base, HALF)])
  pltpu.sync_copy(acc.at[pl.ds(base + HALF, HALF)], zbuf)
  pltpu.sync_copy(zbuf, out.at[c, pl.ds(base + HALF, HALF)])


@functools.partial(
    pl.kernel,
    out_type=jax.ShapeDtypeStruct((NC, NP, 16), jnp.float32),
    mesh=_MESH,
    compiler_params=pltpu.CompilerParams(use_tc_tiling_on_sc=False),
    scratch_types=[
        pltpu.VMEM((NBLK, B), jnp.int32),
        pltpu.VMEM((B, 16), jnp.float32),
        pltpu.VMEM((HALF, 16), jnp.float32),
        pltpu.VMEM_SHARED((NP, 16), jnp.float32),
    ],
)
def _deg_kernel(dstb, out, dst_v, ones_v, zbuf, acc):
  c = lax.axis_index("c")
  s = lax.axis_index("s")
  wid = c * NS + s
  base = s * STRIPE

  one16 = jnp.ones((16,), jnp.float32)
  def orow(i, carry):
    ones_v[i, pl.ds(0, 16)] = one16
    return carry
  lax.fori_loop(0, B, orow, 0)

  _zero_fill(zbuf, HALF, 16)
  pltpu.sync_copy(zbuf, acc.at[pl.ds(base, HALF)])
  pltpu.sync_copy(zbuf, acc.at[pl.ds(base + HALF, HALF)])
  pltpu.sync_copy(dstb.at[wid], dst_v)
  plsc.subcore_barrier()

  def body(j, carry):
    pltpu.sync_copy(ones_v, acc.at[dst_v.at[j]], add=True)
    return carry

  lax.fori_loop(0, NBLK, body, 0)
  plsc.subcore_barrier()

  pltpu.sync_copy(acc.at[pl.ds(base, HALF)], zbuf)
  pltpu.sync_copy(zbuf, out.at[c, pl.ds(base, HALF)])
  pltpu.sync_copy(acc.at[pl.ds(base + HALF, HALF)], zbuf)
  pltpu.sync_copy(zbuf, out.at[c, pl.ds(base + HALF, HALF)])


def _tc1_body(degp_ref, xp_ref, dis_ref, xl_ref, xr_ref):
  deg = degp_ref[0, :, 0:1] + degp_ref[1, :, 0:1]
  rows = lax.broadcasted_iota(jnp.int32, (NP, 1), 0)
  dis = jnp.where(rows < N, lax.rsqrt(jnp.maximum(deg, 1.0)), 0.0)
  dis_ref[...] = dis
  xs = dis * xp_ref[...]
  xl_ref[...] = xs[:, :DH]
  xr_ref[...] = xs[:, DH:]


_tc1 = pl.pallas_call(
    _tc1_body,
    out_shape=[jax.ShapeDtypeStruct((NP, 1), jnp.float32),
               jax.ShapeDtypeStruct((NP, DH), jnp.float32),
               jax.ShapeDtypeStruct((NP, DH), jnp.float32)],
)


def _tc2_body(p_ref, dis_ref, w0_ref, b0_ref, h0_ref, hl_ref, hr_ref):
  dis = dis_ref[...]
  a = jnp.concatenate([p_ref[0], p_ref[1]], axis=1)
  h = jnp.dot(dis * a, w0_ref[...],
              preferred_element_type=jnp.float32) + b0_ref[...]
  h = jnp.maximum(h, 0.0)
  h0_ref[...] = h
  hs = dis * h
  hl_ref[...] = hs[:, :DH]
  hr_ref[...] = hs[:, DH:]


_tc2 = pl.pallas_call(
    _tc2_body,
    out_shape=[jax.ShapeDtypeStruct((NP, 128), jnp.float32),
               jax.ShapeDtypeStruct((NP, DH), jnp.float32),
               jax.ShapeDtypeStruct((NP, DH), jnp.float32)],
)


def _tc3_body(p_ref, dis_ref, w1_ref, g_ref, b1_ref, bt_ref, h0_ref, w2p_ref,
              y2s_ref):
  dis = dis_ref[...]
  a = dis * jnp.concatenate([p_ref[0], p_ref[1]], axis=1)
  g = g_ref[...] * BN_INV
  w1g = w1_ref[...] * g
  b1g = b1_ref[...] * g + bt_ref[...]
  h1 = jnp.dot(a, w1g, preferred_element_type=jnp.float32) + b1g
  h1 = jnp.maximum(h1, 0.0) + h0_ref[...]
  y2s_ref[...] = dis * jnp.dot(h1, w2p_ref[...],
                               preferred_element_type=jnp.float32)


_tc3 = pl.pallas_call(
    _tc3_body,
    out_shape=jax.ShapeDtypeStruct((NP, 16), jnp.float32),
)


def _tc4_body(p_ref, dis_ref, b2p_ref, out_ref):
  z = dis_ref[...] * (p_ref[0] + p_ref[1]) + b2p_ref[...]
  z0 = z[:, 0:1]
  z1 = z[:, 1:2]
  m = jnp.maximum(z0, z1)
  lse = m + jnp.log(jnp.exp(z0 - m) + jnp.exp(z1 - m))
  out_ref[...] = z - lse


_tc4 = pl.pallas_call(
    _tc4_body,
    out_shape=jax.ShapeDtypeStruct((NP, 16), jnp.float32),
)


def kernel(x, edge_index, W0, b0, W1, b1, W2, b2, gamma1, beta1):
  ei = edge_index.astype(jnp.int32)
  loops = jnp.arange(N, dtype=jnp.int32)
  pad_n = EP - (ei.shape[1] + N)
  pad_src = jnp.arange(pad_n, dtype=jnp.int32) % N
  src = jnp.concatenate([ei[0], loops, pad_src])
  pad_dst = N + (jnp.arange(pad_n, dtype=jnp.int32) % (NP - N))
  dst = jnp.concatenate([ei[1], loops, pad_dst])
  srcb_e = src.reshape(NW, NBLK, B)
  dstb_e = dst.reshape(NW, NBLK, B)
  srcb_c = src.reshape(NS, NBLK_CS, BW)
  dstb_c = dst.reshape(NS, NBLK_CS, BW)
  xpad = jnp.pad(x, ((0, NP - N), (0, 0)))

  degp = _deg_kernel(dstb_e)
  dis, xl, xr = _tc1(degp, xpad)
  p1 = _agg128(xl, xr, srcb_c, dstb_c)
  h0, hl, hr = _tc2(p1, dis, W0, b0.reshape(1, -1))
  p2 = _agg128(hl, hr, srcb_c, dstb_c)
  w2p = jnp.pad(W2, ((0, 0), (0, 14)))
  y2s = _tc3(p2, dis, W1, gamma1.reshape(1, -1), b1.reshape(1, -1),
             beta1.reshape(1, -1), h0, w2p)
  p3 = _agg16(y2s, srcb_e, dstb_e)
  b2p = jnp.pad(b2, (0, 14)).reshape(1, 16)
  res = _tc4(p3, dis, b2p)
  return res[:N, :2]
